# Initial kernel scaffold; baseline (speedup 1.0000x reference)
import jax
import jax.numpy as jnp
from jax.experimental import pallas as pl


def kernel(h, edge_index, gate_w, gate_b):
    raise NotImplementedError("write your pallas kernel here")


# speedup vs baseline: 10.1561x; 10.1561x over previous
import functools

import jax
import jax.numpy as jnp
from jax import lax
from jax.experimental import pallas as pl
from jax.experimental.pallas import tpu as pltpu
from jax.experimental.pallas import tpu_sc as plsc

_EPS = 0.3
_NC = 2
_NS = 16
_LANES = 16
_CHUNK = 256
_DCHUNK = 2048
_IDXW = 128


def _rsqrt_newton(x):
    xi = lax.bitcast_convert_type(x, jnp.int32)
    yi = jnp.int32(0x5F3759DF) - (xi >> 1)
    y = lax.bitcast_convert_type(yi, jnp.float32)
    for _ in range(3):
        y = y * (1.5 - 0.5 * x * y * y)
    return y


def _tanh_via_exp(x):
    t = jnp.exp(-2.0 * jnp.abs(x))
    th = (1.0 - t) / (1.0 + t)
    return jnp.where(x < 0.0, -th, th)


def _make_sc_kernel(N, D, E_pad, NPAD):
    RPT = NPAD // _NS
    ept = E_pad // (_NC * _NS)
    n_chunks = ept // _CHUNK
    deg_rows_per_tile = E_pad // _NS // _IDXW
    n_dchunks = deg_rows_per_tile // (_DCHUNK // _IDXW)
    DSUB = _DCHUNK // _IDXW
    CSUB = _CHUNK // _IDXW
    G = _CHUNK // _LANES

    mesh = plsc.VectorSubcoreMesh(
        core_axis_name="c", subcore_axis_name="s",
        num_cores=_NC, num_subcores=_NS)

    @functools.partial(
        pl.kernel,
        out_type=jax.ShapeDtypeStruct((_NC, NPAD, D), jnp.float32),
        mesh=mesh,
        compiler_params=pltpu.CompilerParams(needs_layout_passes=False),
        scratch_types=[
            pltpu.VMEM_SHARED((NPAD,), jnp.float32),
            pltpu.VMEM_SHARED((NPAD,), jnp.float32),
            pltpu.VMEM_SHARED((NPAD,), jnp.float32),
            pltpu.VMEM_SHARED((NPAD,), jnp.float32),
            pltpu.VMEM_SHARED((NPAD, D), jnp.float32),
            pltpu.VMEM((CSUB, _IDXW), jnp.int32),
            pltpu.VMEM((CSUB, _IDXW), jnp.int32),
            pltpu.VMEM((_CHUNK + _LANES,), jnp.float32),
            pltpu.VMEM((_CHUNK,), jnp.float32),
            pltpu.VMEM((_CHUNK,), jnp.float32),
            pltpu.VMEM((_CHUNK,), jnp.float32),
            pltpu.VMEM((_CHUNK,), jnp.float32),
            pltpu.VMEM((_DCHUNK,), jnp.float32),
            pltpu.VMEM((DSUB, _IDXW), jnp.int32),
            pltpu.VMEM((RPT,), jnp.float32),
            pltpu.VMEM((_CHUNK, D), jnp.float32),
            pltpu.SemaphoreType.DMA,
        ],
    )
    def sc_kernel(h_hbm, src_hbm, dst_hbm, a1_hbm, a2_hbm, mm_hbm,
                  deg_sh, a1_sh, a2_sh, d_sh, m_sh, src_v, dst_v, e_v,
                  a1g_v, a2g_v, ddg_v, dsg_v, ones_v, didx_v, z_v,
                  rows_v, sem):
        cid = lax.axis_index("c")
        sid = lax.axis_index("s")
        zeros = jnp.zeros((_LANES,), jnp.float32)
        ones = jnp.ones((_LANES,), jnp.float32)

        @pl.loop(0, RPT // _LANES)
        def _(i):
            z_v[pl.ds(i * _LANES, _LANES)] = zeros

        @pl.loop(0, _DCHUNK // _LANES)
        def _(i):
            ones_v[pl.ds(i * _LANES, _LANES)] = ones

        @pl.loop(0, _IDXW)
        def _(i):
            for k in range(D // _LANES):
                rows_v[i, pl.ds(k * _LANES, _LANES)] = zeros

        base = sid * RPT
        pltpu.sync_copy(z_v, deg_sh.at[pl.ds(base, RPT)])
        for r in range(RPT // _IDXW):
            pltpu.sync_copy(rows_v.at[pl.ds(0, _IDXW), :],
                            m_sh.at[pl.ds(base + r * _IDXW, _IDXW), :])
        pltpu.sync_copy(a1_hbm.at[pl.ds(base, RPT)], a1_sh.at[pl.ds(base, RPT)])
        pltpu.sync_copy(a2_hbm.at[pl.ds(base, RPT)], a2_sh.at[pl.ds(base, RPT)])
        plsc.subcore_barrier()

        drow0 = sid * deg_rows_per_tile

        @pl.loop(0, n_dchunks)
        def _(k):
            row = drow0 + k * DSUB
            pltpu.sync_copy(dst_hbm.at[pl.ds(row, DSUB), :], didx_v)
            for j in range(DSUB):
                pltpu.sync_copy(ones_v.at[pl.ds(j * _IDXW, _IDXW)],
                                deg_sh.at[didx_v.at[j]], add=True)
        plsc.subcore_barrier()

        pltpu.sync_copy(deg_sh.at[pl.ds(base, RPT)], z_v)

        @pl.loop(0, RPT // _LANES)
        def _(i):
            idx = lax.iota(jnp.int32, _LANES) + (base + i * _LANES)
            x = jnp.maximum(z_v[pl.ds(i * _LANES, _LANES)], 1.0)
            y = _rsqrt_newton(x)
            z_v[pl.ds(i * _LANES, _LANES)] = jnp.where(idx >= N, 0.0, y)

        pltpu.sync_copy(z_v, d_sh.at[pl.ds(base, RPT)])
        plsc.subcore_barrier()

        erow0 = (cid * _NS + sid) * (ept // _IDXW)

        @pl.loop(0, n_chunks)
        def _(k):
            row = erow0 + k * CSUB
            pltpu.sync_copy(src_hbm.at[pl.ds(row, CSUB), :], src_v)
            pltpu.sync_copy(dst_hbm.at[pl.ds(row, CSUB), :], dst_v)
            cps = [pltpu.async_copy(h_hbm.at[src_v.at[j]],
                                    rows_v.at[pl.ds(j * _IDXW, _IDXW), :], sem)
                   for j in range(CSUB)]
            for j in range(CSUB):
                sl = pl.ds(j * _IDXW, _IDXW)
                pltpu.sync_copy(a1_sh.at[dst_v.at[j]], a1g_v.at[sl])
                pltpu.sync_copy(a2_sh.at[src_v.at[j]], a2g_v.at[sl])
                pltpu.sync_copy(d_sh.at[dst_v.at[j]], ddg_v.at[sl])
                pltpu.sync_copy(d_sh.at[src_v.at[j]], dsg_v.at[sl])
            for g in range(G):
                sl = pl.ds(g * _LANES, _LANES)
                e = (_tanh_via_exp(a1g_v[sl] + a2g_v[sl])
                     * ddg_v[sl] * dsg_v[sl])
                e_v[sl] = e
            for cp in cps:
                cp.wait()

            @plsc.parallel_loop(0, _CHUNK, unroll=4)
            def _(i):
                es = e_v[pl.ds(i, _LANES)][0]
                for kk in range(D // _LANES):
                    rows_v[i, pl.ds(kk * _LANES, _LANES)] = (
                        rows_v[i, pl.ds(kk * _LANES, _LANES)] * es)

            for j in range(CSUB):
                pltpu.sync_copy(rows_v.at[pl.ds(j * _IDXW, _IDXW), :],
                                m_sh.at[dst_v.at[j]], add=True)
        plsc.subcore_barrier()

        for r in range(RPT // _IDXW):
            pltpu.sync_copy(m_sh.at[pl.ds(base + r * _IDXW, _IDXW), :],
                            mm_hbm.at[cid, pl.ds(base + r * _IDXW, _IDXW), :])

    return sc_kernel


def _pick_bs(n):
    for cand in (1024, 1000, 512, 500, 256, 250, 128, 125, 64, 40, 32, 25,
                 16, 10, 8, 5, 4, 2, 1):
        if n % cand == 0:
            return cand
    return 1


def _gate_proj(h, w2, b2):
    n, d = h.shape
    bs = _pick_bs(n)

    def body(h_ref, w_ref, b_ref, o_ref):
        o_ref[...] = jnp.dot(h_ref[...], w_ref[...],
                             preferred_element_type=jnp.float32) + b_ref[...]

    return pl.pallas_call(
        body,
        grid=(n // bs,),
        in_specs=[pl.BlockSpec((bs, d), lambda i: (i, 0)),
                  pl.BlockSpec((d, 2), lambda i: (0, 0)),
                  pl.BlockSpec((1, 2), lambda i: (0, 0))],
        out_specs=pl.BlockSpec((bs, 2), lambda i: (i, 0)),
        out_shape=jax.ShapeDtypeStruct((n, 2), jnp.float32),
    )(h, w2, b2)


def _combine(h, mm):
    n, d = h.shape
    bs = _pick_bs(n)

    def body(h_ref, m0_ref, m1_ref, o_ref):
        o_ref[...] = jnp.maximum(
            _EPS * h_ref[...] + m0_ref[0] + m1_ref[0], 0.0)

    return pl.pallas_call(
        body,
        grid=(n // bs,),
        in_specs=[pl.BlockSpec((bs, d), lambda i: (i, 0)),
                  pl.BlockSpec((1, bs, d), lambda i: (0, i, 0)),
                  pl.BlockSpec((1, bs, d), lambda i: (1, i, 0))],
        out_specs=pl.BlockSpec((bs, d), lambda i: (i, 0)),
        out_shape=jax.ShapeDtypeStruct((n, d), jnp.float32),
    )(h, mm, mm)


def kernel(h, edge_index, gate_w, gate_b):
    n, d = h.shape
    e = edge_index.shape[1]

    npad = -((n + 1) // -(_NS * _LANES)) * (_NS * _LANES)
    estep = max(_NC * _NS * _CHUNK, _NS * _DCHUNK)
    e_pad = -(e // -estep) * estep

    src = edge_index[0]
    dst = edge_index[1]
    pad = e_pad - e
    srcp = jnp.concatenate(
        [src, jnp.zeros((pad,), jnp.int32)]).reshape(e_pad // _IDXW, _IDXW)
    dstp = jnp.concatenate(
        [dst, jnp.full((pad,), n, jnp.int32)]).reshape(e_pad // _IDXW, _IDXW)

    w_dst = gate_w[:d, 0]
    w_src = gate_w[d:, 0]
    w2 = jnp.stack([w_dst, w_src], axis=1)
    b2 = jnp.stack([gate_b[0], jnp.zeros((), jnp.float32)]).reshape(1, 2)

    a = _gate_proj(h, w2, b2)
    a1 = jnp.pad(a[:, 0], (0, npad - n))
    a2 = jnp.pad(a[:, 1], (0, npad - n))

    mm = _make_sc_kernel(n, d, e_pad, npad)(h, srcp, dstp, a1, a2)
    return _combine(h, mm)


# speedup vs baseline: 11.8833x; 1.1701x over previous
import functools

import jax
import jax.numpy as jnp
from jax import lax
from jax.experimental import pallas as pl
from jax.experimental.pallas import tpu as pltpu
from jax.experimental.pallas import tpu_sc as plsc

_EPS = 0.3
_NC = 2
_NS = 16
_LANES = 16
_CHUNK = 128
_DCHUNK = 2048
_IDXW = 128


def _rsqrt_newton(x):
    xi = lax.bitcast_convert_type(x, jnp.int32)
    yi = jnp.int32(0x5F3759DF) - (xi >> 1)
    y = lax.bitcast_convert_type(yi, jnp.float32)
    for _ in range(3):
        y = y * (1.5 - 0.5 * x * y * y)
    return y


def _tanh_via_exp(x):
    t = jnp.exp(-2.0 * jnp.abs(x))
    th = (1.0 - t) / (1.0 + t)
    return jnp.where(x < 0.0, -th, th)


def _make_sc_kernel(N, D, E_pad, NPAD):
    RPT = NPAD // _NS
    ept = E_pad // (_NC * _NS)
    n_chunks = ept // _CHUNK
    deg_rows_per_tile = E_pad // _NS // _IDXW
    n_dchunks = deg_rows_per_tile // (_DCHUNK // _IDXW)
    DSUB = _DCHUNK // _IDXW
    CSUB = _CHUNK // _IDXW
    G = _CHUNK // _LANES

    mesh = plsc.VectorSubcoreMesh(
        core_axis_name="c", subcore_axis_name="s",
        num_cores=_NC, num_subcores=_NS)

    @functools.partial(
        pl.kernel,
        out_type=jax.ShapeDtypeStruct((_NC, NPAD, D), jnp.float32),
        mesh=mesh,
        compiler_params=pltpu.CompilerParams(needs_layout_passes=False),
        scratch_types=[
            pltpu.VMEM_SHARED((NPAD,), jnp.float32),
            pltpu.VMEM_SHARED((NPAD,), jnp.float32),
            pltpu.VMEM_SHARED((NPAD,), jnp.float32),
            pltpu.VMEM_SHARED((NPAD,), jnp.float32),
            pltpu.VMEM_SHARED((NPAD, D), jnp.float32),
            pltpu.VMEM((2, _CHUNK), jnp.int32),
            pltpu.VMEM((2, _CHUNK), jnp.int32),
            pltpu.VMEM((2, _CHUNK + _LANES), jnp.float32),
            pltpu.VMEM((2, _CHUNK), jnp.float32),
            pltpu.VMEM((2, _CHUNK), jnp.float32),
            pltpu.VMEM((2, _CHUNK), jnp.float32),
            pltpu.VMEM((2, _CHUNK), jnp.float32),
            pltpu.VMEM((_DCHUNK,), jnp.float32),
            pltpu.VMEM((DSUB, _IDXW), jnp.int32),
            pltpu.VMEM((RPT,), jnp.float32),
            pltpu.VMEM((2, _CHUNK, D), jnp.float32),
            pltpu.SemaphoreType.DMA,
            pltpu.SemaphoreType.DMA,
        ],
    )
    def sc_kernel(h_hbm, src_hbm, dst_hbm, a1_hbm, a2_hbm, mm_hbm,
                  deg_sh, a1_sh, a2_sh, d_sh, m_sh, src_v, dst_v, e_v,
                  a1g_v, a2g_v, ddg_v, dsg_v, ones_v, didx_v, z_v,
                  rows_v, sem0, sem1):
        sems = (sem0, sem1)
        cid = lax.axis_index("c")
        sid = lax.axis_index("s")
        zeros = jnp.zeros((_LANES,), jnp.float32)
        ones = jnp.ones((_LANES,), jnp.float32)

        @pl.loop(0, RPT // _LANES)
        def _(i):
            z_v[pl.ds(i * _LANES, _LANES)] = zeros

        @pl.loop(0, _DCHUNK // _LANES)
        def _(i):
            ones_v[pl.ds(i * _LANES, _LANES)] = ones

        @pl.loop(0, _IDXW)
        def _(i):
            for k in range(D // _LANES):
                rows_v[0, i, pl.ds(k * _LANES, _LANES)] = zeros

        base = sid * RPT
        pltpu.sync_copy(z_v, deg_sh.at[pl.ds(base, RPT)])
        for r in range(RPT // _IDXW):
            pltpu.sync_copy(rows_v.at[0],
                            m_sh.at[pl.ds(base + r * _IDXW, _IDXW), :])
        pltpu.sync_copy(a1_hbm.at[pl.ds(base, RPT)], a1_sh.at[pl.ds(base, RPT)])
        pltpu.sync_copy(a2_hbm.at[pl.ds(base, RPT)], a2_sh.at[pl.ds(base, RPT)])
        plsc.subcore_barrier()

        drow0 = sid * deg_rows_per_tile

        @pl.loop(0, n_dchunks)
        def _(k):
            row = drow0 + k * DSUB
            pltpu.sync_copy(dst_hbm.at[pl.ds(row, DSUB), :], didx_v)
            for j in range(DSUB):
                pltpu.sync_copy(ones_v.at[pl.ds(j * _IDXW, _IDXW)],
                                deg_sh.at[didx_v.at[j]], add=True)
        plsc.subcore_barrier()

        pltpu.sync_copy(deg_sh.at[pl.ds(base, RPT)], z_v)

        @pl.loop(0, RPT // _LANES)
        def _(i):
            idx = lax.iota(jnp.int32, _LANES) + (base + i * _LANES)
            x = jnp.maximum(z_v[pl.ds(i * _LANES, _LANES)], 1.0)
            y = _rsqrt_newton(x)
            z_v[pl.ds(i * _LANES, _LANES)] = jnp.where(idx >= N, 0.0, y)

        pltpu.sync_copy(z_v, d_sh.at[pl.ds(base, RPT)])
        plsc.subcore_barrier()

        erow0 = (cid * _NS + sid) * n_chunks

        def _prefetch(krow, nb):
            pltpu.sync_copy(src_hbm.at[krow], src_v.at[nb])
            pltpu.sync_copy(dst_hbm.at[krow], dst_v.at[nb])
            pltpu.async_copy(h_hbm.at[src_v.at[nb]], rows_v.at[nb], sems[nb])
            pltpu.sync_copy(a1_sh.at[dst_v.at[nb]], a1g_v.at[nb])
            pltpu.sync_copy(a2_sh.at[src_v.at[nb]], a2g_v.at[nb])
            pltpu.sync_copy(d_sh.at[dst_v.at[nb]], ddg_v.at[nb])
            pltpu.sync_copy(d_sh.at[src_v.at[nb]], dsg_v.at[nb])
            for g in range(G):
                sl = pl.ds(g * _LANES, _LANES)
                e_v[nb, sl] = (_tanh_via_exp(a1g_v[nb, sl] + a2g_v[nb, sl])
                               * ddg_v[nb, sl] * dsg_v[nb, sl])

        _prefetch(erow0, 0)

        @pl.loop(0, n_chunks // 2)
        def _(p):
            for b in range(2):
                k = p * 2 + b
                nb = 1 - b
                nk = jnp.minimum(k + 1, n_chunks - 1)
                _prefetch(erow0 + nk, nb)
                pltpu.make_async_copy(h_hbm.at[src_v.at[b]],
                                      rows_v.at[b], sems[b]).wait()

                @plsc.parallel_loop(0, _CHUNK, unroll=4)
                def _(i):
                    es = e_v[b, pl.ds(i, _LANES)][0]
                    for kk in range(D // _LANES):
                        rows_v[b, i, pl.ds(kk * _LANES, _LANES)] = (
                            rows_v[b, i, pl.ds(kk * _LANES, _LANES)] * es)

                pltpu.sync_copy(rows_v.at[b], m_sh.at[dst_v.at[b]], add=True)

        pltpu.make_async_copy(h_hbm.at[src_v.at[0]], rows_v.at[0],
                              sems[0]).wait()
        plsc.subcore_barrier()

        for r in range(RPT // _IDXW):
            pltpu.sync_copy(m_sh.at[pl.ds(base + r * _IDXW, _IDXW), :],
                            mm_hbm.at[cid, pl.ds(base + r * _IDXW, _IDXW), :])

    return sc_kernel


def _pick_bs(n):
    for cand in (1024, 1000, 512, 500, 256, 250, 128, 125, 64, 40, 32, 25,
                 16, 10, 8, 5, 4, 2, 1):
        if n % cand == 0:
            return cand
    return 1


def _gate_proj(h, w2, b2):
    n, d = h.shape
    bs = _pick_bs(n)

    def body(h_ref, w_ref, b_ref, o_ref):
        o_ref[...] = jnp.dot(h_ref[...], w_ref[...],
                             preferred_element_type=jnp.float32) + b_ref[...]

    return pl.pallas_call(
        body,
        grid=(n // bs,),
        in_specs=[pl.BlockSpec((bs, d), lambda i: (i, 0)),
                  pl.BlockSpec((d, 2), lambda i: (0, 0)),
                  pl.BlockSpec((1, 2), lambda i: (0, 0))],
        out_specs=pl.BlockSpec((bs, 2), lambda i: (i, 0)),
        out_shape=jax.ShapeDtypeStruct((n, 2), jnp.float32),
    )(h, w2, b2)


def _combine(h, mm):
    n, d = h.shape
    bs = _pick_bs(n)

    def body(h_ref, m0_ref, m1_ref, o_ref):
        o_ref[...] = jnp.maximum(
            _EPS * h_ref[...] + m0_ref[0] + m1_ref[0], 0.0)

    return pl.pallas_call(
        body,
        grid=(n // bs,),
        in_specs=[pl.BlockSpec((bs, d), lambda i: (i, 0)),
                  pl.BlockSpec((1, bs, d), lambda i: (0, i, 0)),
                  pl.BlockSpec((1, bs, d), lambda i: (1, i, 0))],
        out_specs=pl.BlockSpec((bs, d), lambda i: (i, 0)),
        out_shape=jax.ShapeDtypeStruct((n, d), jnp.float32),
    )(h, mm, mm)


def kernel(h, edge_index, gate_w, gate_b):
    n, d = h.shape
    e = edge_index.shape[1]

    npad = -((n + 1) // -(_NS * _LANES)) * (_NS * _LANES)
    estep = max(_NC * _NS * _CHUNK, _NS * _DCHUNK)
    e_pad = -(e // -estep) * estep

    src = edge_index[0]
    dst = edge_index[1]
    pad = e_pad - e
    srcp = jnp.concatenate(
        [src, jnp.zeros((pad,), jnp.int32)]).reshape(e_pad // _IDXW, _IDXW)
    dstp = jnp.concatenate(
        [dst, jnp.full((pad,), n, jnp.int32)]).reshape(e_pad // _IDXW, _IDXW)

    w_dst = gate_w[:d, 0]
    w_src = gate_w[d:, 0]
    w2 = jnp.stack([w_dst, w_src], axis=1)
    b2 = jnp.stack([gate_b[0], jnp.zeros((), jnp.float32)]).reshape(1, 2)

    a = _gate_proj(h, w2, b2)
    a1 = jnp.pad(a[:, 0], (0, npad - n))
    a2 = jnp.pad(a[:, 1], (0, npad - n))

    mm = _make_sc_kernel(n, d, e_pad, npad)(h, srcp, dstp, a1, a2)
    return _combine(h, mm)
